# Initial kernel scaffold; baseline (speedup 1.0000x reference)
import jax
import jax.numpy as jnp
from jax.experimental import pallas as pl


def kernel(x, delays):
    raise NotImplementedError("write your pallas kernel here")


# speedup vs baseline: 8.2848x; 8.2848x over previous
import jax
import jax.numpy as jnp
from jax.experimental import pallas as pl

_T = 100
_BBLK = 8


def _body(x_ref, d_ref, o_ref):
    st = ((1.0 - x_ref[...] * d_ref[...]) * (_T - 1)).astype(jnp.int32)
    st = jnp.clip(st, 0, _T - 1)
    t = jax.lax.broadcasted_iota(jnp.int32, (_BBLK, _T, st.shape[-1]), 1)
    o_ref[...] = (t == st[:, None, :]).astype(jnp.float32)


def kernel(x, delays):
    b, f = x.shape
    return pl.pallas_call(
        _body,
        grid=(b // _BBLK,),
        in_specs=[
            pl.BlockSpec((_BBLK, f), lambda i: (i, 0)),
            pl.BlockSpec((1, f), lambda i: (0, 0)),
        ],
        out_specs=pl.BlockSpec((_BBLK, _T, f), lambda i: (i, 0, 0)),
        out_shape=jax.ShapeDtypeStruct((b, _T, f), jnp.float32),
    )(x, delays[None, :])


# speedup vs baseline: 8.6041x; 1.0386x over previous
import jax
import jax.numpy as jnp
from jax.experimental import pallas as pl

_T = 100
_BBLK = 32


def _body(x_ref, d_ref, o_ref):
    st = ((1.0 - x_ref[...] * d_ref[...]) * (_T - 1)).astype(jnp.int32)
    st = jnp.clip(st, 0, _T - 1)
    t = jax.lax.broadcasted_iota(jnp.int32, (_BBLK, _T, st.shape[-1]), 1)
    o_ref[...] = (t == st[:, None, :]).astype(jnp.float32)


def kernel(x, delays):
    b, f = x.shape
    return pl.pallas_call(
        _body,
        grid=(b // _BBLK,),
        in_specs=[
            pl.BlockSpec((_BBLK, f), lambda i: (i, 0)),
            pl.BlockSpec((1, f), lambda i: (0, 0)),
        ],
        out_specs=pl.BlockSpec((_BBLK, _T, f), lambda i: (i, 0, 0)),
        out_shape=jax.ShapeDtypeStruct((b, _T, f), jnp.float32),
    )(x, delays[None, :])
